# Initial kernel scaffold; baseline (speedup 1.0000x reference)
import jax
import jax.numpy as jnp
from jax.experimental import pallas as pl


def kernel(static, known_real, known_categorical, observed, static_tables, known_cat_tables, W_real, b_real, W_obs, b_obs):
    raise NotImplementedError("write your pallas kernel here")


# speedup vs baseline: 1.4107x; 1.4107x over previous
import functools

import jax
import jax.numpy as jnp
from jax import lax
from jax.experimental import pallas as pl
from jax.experimental.pallas import tpu as pltpu
from jax.experimental.pallas import tpu_sc as plsc

B = 1024
T = 50
H = 64
N_STATIC = 8
STATIC_VOCAB = 100000
N_KNOWN_CAT = 4
KNOWN_CAT_VOCAB = 1000
N_REAL = 4
N_OBS = 6

P = B * T
S_ROWS = B * N_STATIC
K_ROWS = P * (N_REAL + N_KNOWN_CAT)

NW = 32
CHUNK = 80
POS_PER_W = P // NW
N_CHUNKS = POS_PER_W // CHUNK
SPW = S_ROWS // NW


def _sc_gather(kc_tbl, st_tbl, gidx_hbm, dest_hbm, stidx_hbm):
    mesh = plsc.VectorSubcoreMesh(core_axis_name="c", subcore_axis_name="s",
                                  num_cores=2, num_subcores=16)

    @functools.partial(
        pl.kernel,
        out_type=(
            jax.ShapeDtypeStruct((K_ROWS, H), jnp.float32),
            jax.ShapeDtypeStruct((S_ROWS, H), jnp.float32),
        ),
        mesh=mesh,
        compiler_params=pltpu.CompilerParams(use_tc_tiling_on_sc=False),
        scratch_types=[
            pltpu.VMEM((4, CHUNK), jnp.int32),
            pltpu.VMEM((4, CHUNK), jnp.int32),
            pltpu.VMEM((4 * CHUNK, H), jnp.float32),
            pltpu.VMEM((2, 128), jnp.int32),
            pltpu.VMEM((SPW, H), jnp.float32),
            pltpu.SemaphoreType.DMA,
            pltpu.SemaphoreType.DMA,
        ],
    )
    def k(kc_ref, st_ref, gidx_ref, dest_ref, stidx_ref,
          kp_out, st_out, gidx_v, sidx_v, cat_buf, stidx_v, st_buf,
          sem_g, sem_s):
        nc = 2
        wid = lax.axis_index("s") * nc + lax.axis_index("c")

        pltpu.sync_copy(stidx_ref.at[pl.ds(wid * 2, 2)], stidx_v)
        cps = [
            pltpu.async_copy(st_ref.at[stidx_v.at[g]],
                             st_buf.at[pl.ds(g * 128, 128)], sem_s)
            for g in range(2)
        ]
        for cp in cps:
            cp.wait()
        pltpu.sync_copy(st_buf, st_out.at[pl.ds(wid * SPW, SPW)])

        def chunk_body(c, carry):
            cid = wid * N_CHUNKS + c
            pltpu.sync_copy(gidx_ref.at[pl.ds(cid * 4, 4)], gidx_v)
            pltpu.sync_copy(dest_ref.at[pl.ds(cid * 4, 4)], sidx_v)
            cps_g = [
                pltpu.async_copy(kc_ref.at[gidx_v.at[g]],
                                 cat_buf.at[pl.ds(g * CHUNK, CHUNK)], sem_g)
                for g in range(4)
            ]
            for cp in cps_g:
                cp.wait()
            cps_s = [
                pltpu.async_copy(cat_buf.at[pl.ds(g * CHUNK, CHUNK)],
                                 kp_out.at[sidx_v.at[g]], sem_s)
                for g in range(4)
            ]
            for cp in cps_s:
                cp.wait()
            return carry

        lax.fori_loop(0, N_CHUNKS, chunk_body, 0)

    return k(kc_tbl, st_tbl, gidx_hbm, dest_hbm, stidx_hbm)


def _tc_dense(kp, known_real_f, observed_f, W_real, b_real, W_obs, b_obs):
    PP = 1024
    grid = (P // PP,)

    def body(kp_ref, kr_ref, obs_ref, wr_ref, br_ref, wo_ref, bo_ref,
             out_k_ref, out_o_ref):
        del kp_ref
        kr = kr_ref[...]
        wr = wr_ref[...]
        br = br_ref[...]
        for i in range(N_REAL):
            out_k_ref[:, pl.ds(i * H, H)] = (
                kr[:, i:i + 1] * wr[i:i + 1, :] + br[i:i + 1, :])
        ob = obs_ref[...]
        wo = wo_ref[...]
        bo = bo_ref[...]
        for i in range(N_OBS):
            out_o_ref[:, pl.ds(i * H, H)] = (
                ob[:, i:i + 1] * wo[i:i + 1, :] + bo[i:i + 1, :])

    out_k, out_o = pl.pallas_call(
        body,
        grid=grid,
        in_specs=[
            pl.BlockSpec(memory_space=pl.ANY),
            pl.BlockSpec((PP, N_REAL), lambda i: (i, 0)),
            pl.BlockSpec((PP, N_OBS), lambda i: (i, 0)),
            pl.BlockSpec((N_REAL, H), lambda i: (0, 0)),
            pl.BlockSpec((N_REAL, H), lambda i: (0, 0)),
            pl.BlockSpec((N_OBS, H), lambda i: (0, 0)),
            pl.BlockSpec((N_OBS, H), lambda i: (0, 0)),
        ],
        out_specs=[
            pl.BlockSpec((PP, N_REAL * H), lambda i: (i, 0)),
            pl.BlockSpec((PP, N_OBS * H), lambda i: (i, 0)),
        ],
        out_shape=[
            jax.ShapeDtypeStruct((P, (N_REAL + N_KNOWN_CAT) * H), jnp.float32),
            jax.ShapeDtypeStruct((P, N_OBS * H), jnp.float32),
        ],
        input_output_aliases={0: 0},
    )(kp, known_real_f, observed_f, W_real, b_real, W_obs, b_obs)
    return out_k, out_o


def kernel(static, known_real, known_categorical, observed,
           static_tables, known_cat_tables, W_real, b_real, W_obs, b_obs):
    st_tbl = static_tables.reshape(N_STATIC * STATIC_VOCAB, H)
    kc_tbl = known_cat_tables.reshape(N_KNOWN_CAT * KNOWN_CAT_VOCAB, H)

    st_idx = (static + jnp.arange(N_STATIC, dtype=jnp.int32) * STATIC_VOCAB)
    st_idx = st_idx.reshape(S_ROWS // 128, 128)

    kc_idx = (known_categorical.reshape(P, N_KNOWN_CAT)
              + jnp.arange(N_KNOWN_CAT, dtype=jnp.int32) * KNOWN_CAT_VOCAB)
    gidx = kc_idx.reshape(P * N_KNOWN_CAT // CHUNK, CHUNK)

    dest = (jnp.arange(P, dtype=jnp.int32)[:, None] * (N_REAL + N_KNOWN_CAT)
            + N_REAL + jnp.arange(N_KNOWN_CAT, dtype=jnp.int32)[None, :])
    dest = dest.reshape(P * N_KNOWN_CAT // CHUNK, CHUNK)

    kp, st_rows = _sc_gather(kc_tbl, st_tbl, gidx, dest, st_idx)

    out_k, out_o = _tc_dense(
        kp.reshape(P, (N_REAL + N_KNOWN_CAT) * H),
        known_real.reshape(P, N_REAL),
        observed.reshape(P, N_OBS),
        W_real, b_real, W_obs, b_obs,
    )

    static_emb = st_rows.reshape(B, N_STATIC, H)
    known_emb = out_k.reshape(B, T, N_REAL + N_KNOWN_CAT, H)
    observed_emb = out_o.reshape(B, T, N_OBS, H)
    return static_emb, known_emb, observed_emb


# speedup vs baseline: 2.9981x; 2.1252x over previous
import functools

import jax
import jax.numpy as jnp
from jax import lax
from jax.experimental import pallas as pl
from jax.experimental.pallas import tpu as pltpu
from jax.experimental.pallas import tpu_sc as plsc

B = 1024
T = 50
H = 64
N_STATIC = 8
STATIC_VOCAB = 100000
N_KNOWN_CAT = 4
KNOWN_CAT_VOCAB = 1000
N_REAL = 4
N_OBS = 6
N_KNOWN = N_REAL + N_KNOWN_CAT

P = B * T
S_ROWS = B * N_STATIC
NW = 32
SPW = S_ROWS // NW


def _sc_static(st_tbl, stidx_hbm):
    mesh = plsc.VectorSubcoreMesh(core_axis_name="c", subcore_axis_name="s",
                                  num_cores=2, num_subcores=16)

    @functools.partial(
        pl.kernel,
        out_type=jax.ShapeDtypeStruct((S_ROWS, H), jnp.float32),
        mesh=mesh,
        compiler_params=pltpu.CompilerParams(use_tc_tiling_on_sc=False),
        scratch_types=[
            pltpu.VMEM((2, 128), jnp.int32),
            pltpu.VMEM((SPW, H), jnp.float32),
            pltpu.SemaphoreType.DMA,
        ],
    )
    def k(st_ref, stidx_ref, st_out, stidx_v, st_buf, sem):
        wid = lax.axis_index("s") * 2 + lax.axis_index("c")
        pltpu.sync_copy(stidx_ref.at[pl.ds(wid * 2, 2)], stidx_v)
        cps = [
            pltpu.async_copy(st_ref.at[stidx_v.at[g]],
                             st_buf.at[pl.ds(g * 128, 128)], sem)
            for g in range(2)
        ]
        for cp in cps:
            cp.wait()
        pltpu.sync_copy(st_buf, st_out.at[pl.ds(wid * SPW, SPW)])

    return k(st_tbl, stidx_hbm)


def _sc_cat(kctbl_t, kc_t):
    mesh = plsc.VectorSubcoreMesh(core_axis_name="c", subcore_axis_name="s",
                                  num_cores=2, num_subcores=16)
    HS = 8

    @functools.partial(
        pl.kernel,
        out_type=jax.ShapeDtypeStruct((T, N_KNOWN, H, B), jnp.float32),
        mesh=mesh,
        compiler_params=pltpu.CompilerParams(use_tc_tiling_on_sc=True,
                                             needs_layout_passes=False),
        scratch_types=[
            pltpu.VMEM((HS, KNOWN_CAT_VOCAB), jnp.float32),
            pltpu.VMEM((B,), jnp.int32),
            pltpu.VMEM((2, HS, B), jnp.float32),
            pltpu.SemaphoreType.DMA,
            pltpu.SemaphoreType.DMA,
        ],
    )
    def k(tbl_ref, idx_ref, out_ref, slab_v, idx_v, stage_v, sem_w0, sem_w1):
        wid = lax.axis_index("s") * 2 + lax.axis_index("c")
        fc = wid // 8
        h0 = (wid % 8) * HS
        pltpu.sync_copy(tbl_ref.at[fc, pl.ds(h0, HS)], slab_v)
        sems = [sem_w0, sem_w1]

        def t_body(t, carry):
            buf = lax.rem(t, 2)
            pltpu.sync_copy(idx_ref.at[t, fc], idx_v)
            for h in range(HS):
                hv = jnp.full((16,), h, jnp.int32)

                def grp(g, c2):
                    iv = idx_v[pl.ds(g * 16, 16)]
                    vals = plsc.load_gather(slab_v, [hv, iv])
                    stage_v[buf, h, pl.ds(g * 16, 16)] = vals
                    return c2

                lax.fori_loop(0, B // 16, grp, 0, unroll=4)
            @pl.when(t >= 2)
            def _():
                pltpu.make_async_copy(
                    stage_v.at[buf], out_ref.at[t - 2, N_REAL + fc,
                                                pl.ds(h0, HS)],
                    sems[0]).wait()

            pltpu.make_async_copy(
                stage_v.at[buf],
                out_ref.at[t, N_REAL + fc, pl.ds(h0, HS)],
                sems[0]).start()
            return carry

        lax.fori_loop(0, T, t_body, 0)
        for t in (T - 2, T - 1):
            pltpu.make_async_copy(
                stage_v.at[t % 2],
                out_ref.at[t, N_REAL + fc, pl.ds(h0, HS)],
                sems[0]).wait()

    return k(kctbl_t, kc_t)


def _tc_dense(kp_t, kr_t, obs_t, wrt, brt, wot, bot):
    grid = (T,)

    def body(kp_ref, kr_ref, obs_ref, wrt_ref, brt_ref, wot_ref, bot_ref,
             outk_ref, outo_ref):
        del kp_ref
        kr = kr_ref[...]
        wr = wrt_ref[...]
        br = brt_ref[...]
        for f in range(N_REAL):
            outk_ref[0, f] = (wr[:, f:f + 1] * kr[0, f][None, :]
                              + br[:, f:f + 1])
        ob = obs_ref[...]
        wo = wot_ref[...]
        bo = bot_ref[...]
        for f in range(N_OBS):
            outo_ref[0, f] = (wo[:, f:f + 1] * ob[f][None, :]
                              + bo[:, f:f + 1])

    out_k, out_o = pl.pallas_call(
        body,
        grid=grid,
        in_specs=[
            pl.BlockSpec(memory_space=pl.ANY),
            pl.BlockSpec((1, N_REAL, B), lambda t: (t, 0, 0)),
            pl.BlockSpec((N_OBS, B), lambda t: (0, t)),
            pl.BlockSpec((H, N_REAL), lambda t: (0, 0)),
            pl.BlockSpec((H, N_REAL), lambda t: (0, 0)),
            pl.BlockSpec((H, N_OBS), lambda t: (0, 0)),
            pl.BlockSpec((H, N_OBS), lambda t: (0, 0)),
        ],
        out_specs=[
            pl.BlockSpec((1, N_REAL, H, B), lambda t: (t, 0, 0, 0)),
            pl.BlockSpec((1, N_OBS, H, B), lambda t: (t, 0, 0, 0)),
        ],
        out_shape=[
            jax.ShapeDtypeStruct((T, N_KNOWN, H, B), jnp.float32),
            jax.ShapeDtypeStruct((T, N_OBS, H, B), jnp.float32),
        ],
        input_output_aliases={0: 0},
    )(kp_t, kr_t, obs_t, wrt, brt, wot, bot)
    return out_k, out_o


def kernel(static, known_real, known_categorical, observed,
           static_tables, known_cat_tables, W_real, b_real, W_obs, b_obs):
    kctbl_t = known_cat_tables.transpose(0, 2, 1)
    kc_t = known_categorical.transpose(1, 2, 0)
    kr_t = known_real.transpose(1, 2, 0)
    obs_t = observed.transpose(2, 1, 0).reshape(N_OBS, P)
    wrt = W_real.T
    brt = b_real.T
    wot = W_obs.T
    bot = b_obs.T

    st_tbl = static_tables.reshape(N_STATIC * STATIC_VOCAB, H)
    st_idx = (static + jnp.arange(N_STATIC, dtype=jnp.int32) * STATIC_VOCAB)
    st_idx = st_idx.reshape(S_ROWS // 128, 128)

    kp_t = _sc_cat(kctbl_t, kc_t)
    st_rows = _sc_static(st_tbl, st_idx)

    out_k, out_o = _tc_dense(kp_t, kr_t, obs_t, wrt, brt, wot, bot)

    static_emb = st_rows.reshape(B, N_STATIC, H)
    known_emb = out_k.transpose(3, 0, 1, 2)
    observed_emb = out_o.transpose(3, 0, 1, 2)
    return static_emb, known_emb, observed_emb


# speedup vs baseline: 3.1083x; 1.0368x over previous
import functools

import jax
import jax.numpy as jnp
from jax import lax
from jax.experimental import pallas as pl
from jax.experimental.pallas import tpu as pltpu
from jax.experimental.pallas import tpu_sc as plsc

B = 1024
T = 50
H = 64
N_STATIC = 8
STATIC_VOCAB = 100000
N_KNOWN_CAT = 4
KNOWN_CAT_VOCAB = 1000
N_REAL = 4
N_OBS = 6
N_KNOWN = N_REAL + N_KNOWN_CAT

P = B * T
S_ROWS = B * N_STATIC
NW = 32
SPW = S_ROWS // NW


def _sc_static(st_tbl, stidx_hbm):
    mesh = plsc.VectorSubcoreMesh(core_axis_name="c", subcore_axis_name="s",
                                  num_cores=2, num_subcores=16)

    @functools.partial(
        pl.kernel,
        out_type=jax.ShapeDtypeStruct((S_ROWS, H), jnp.float32),
        mesh=mesh,
        compiler_params=pltpu.CompilerParams(use_tc_tiling_on_sc=False),
        scratch_types=[
            pltpu.VMEM((2, 128), jnp.int32),
            pltpu.VMEM((SPW, H), jnp.float32),
            pltpu.SemaphoreType.DMA,
        ],
    )
    def k(st_ref, stidx_ref, st_out, stidx_v, st_buf, sem):
        wid = lax.axis_index("s") * 2 + lax.axis_index("c")
        pltpu.sync_copy(stidx_ref.at[pl.ds(wid * 2, 2)], stidx_v)
        cps = [
            pltpu.async_copy(st_ref.at[stidx_v.at[g]],
                             st_buf.at[pl.ds(g * 128, 128)], sem)
            for g in range(2)
        ]
        for cp in cps:
            cp.wait()
        pltpu.sync_copy(st_buf, st_out.at[pl.ds(wid * SPW, SPW)])

    return k(st_tbl, stidx_hbm)


def _sc_cat(kctbl_t, kc_t):
    mesh = plsc.VectorSubcoreMesh(core_axis_name="c", subcore_axis_name="s",
                                  num_cores=2, num_subcores=16)
    HS = 8

    VP = 1024

    @functools.partial(
        pl.kernel,
        out_type=jax.ShapeDtypeStruct((T, N_KNOWN, H, B), jnp.float32),
        mesh=mesh,
        compiler_params=pltpu.CompilerParams(use_tc_tiling_on_sc=True,
                                             needs_layout_passes=False),
        scratch_types=[
            pltpu.VMEM((HS, KNOWN_CAT_VOCAB), jnp.float32),
            pltpu.VMEM((HS * VP,), jnp.float32),
            pltpu.VMEM((2, B), jnp.int32),
            pltpu.VMEM((2, HS, B), jnp.float32),
            pltpu.SemaphoreType.DMA,
            pltpu.SemaphoreType.DMA,
        ],
    )
    def k(tbl_ref, idx_ref, out_ref, slab2_v, slab_v, idx_v, stage_v,
          sem_i, sem_w):
        wid = lax.axis_index("s") * 2 + lax.axis_index("c")
        fc = wid // 8
        h0 = (wid % 8) * HS
        pltpu.sync_copy(tbl_ref.at[fc, pl.ds(h0, HS)], slab2_v)
        for h in range(HS):
            def rearr(c, carry):
                off = c * 16
                slab_v[pl.ds(h * VP + off, 16)] = slab2_v[h, pl.ds(off, 16)]
                return carry
            lax.fori_loop(0, KNOWN_CAT_VOCAB // 16, rearr, 0, unroll=8)
            tail = KNOWN_CAT_VOCAB - 16
            slab_v[pl.ds(h * VP + tail, 16)] = slab2_v[h, pl.ds(tail, 16)]

        pltpu.make_async_copy(idx_ref.at[0, fc], idx_v.at[0], sem_i).start()

        def t_body(t, carry):
            buf = lax.rem(t, 2)
            pltpu.make_async_copy(idx_ref.at[t, fc], idx_v.at[buf],
                                  sem_i).wait()
            @pl.when(t + 1 < T)
            def _():
                pltpu.make_async_copy(idx_ref.at[t + 1, fc],
                                      idx_v.at[1 - buf], sem_i).start()

            for g in range(B // 16):
                iv = idx_v[buf, pl.ds(g * 16, 16)]
                for h in range(HS):
                    av = iv + (h * VP) if h else iv
                    vals = plsc.load_gather(slab_v, [av])
                    stage_v[buf, h, pl.ds(g * 16, 16)] = vals

            @pl.when(t >= 2)
            def _():
                pltpu.make_async_copy(
                    stage_v.at[buf],
                    out_ref.at[t - 2, N_REAL + fc, pl.ds(h0, HS)],
                    sem_w).wait()

            pltpu.make_async_copy(
                stage_v.at[buf],
                out_ref.at[t, N_REAL + fc, pl.ds(h0, HS)],
                sem_w).start()
            return carry

        lax.fori_loop(0, T, t_body, 0)
        for t in (T - 2, T - 1):
            pltpu.make_async_copy(
                stage_v.at[t % 2],
                out_ref.at[t, N_REAL + fc, pl.ds(h0, HS)],
                sem_w).wait()

    return k(kctbl_t, kc_t)


def _tc_dense(kp_t, kr_t, obs_t, wrt, brt, wot, bot):
    grid = (T,)

    def body(kp_ref, kr_ref, obs_ref, wrt_ref, brt_ref, wot_ref, bot_ref,
             outk_ref, outo_ref):
        del kp_ref
        kr = kr_ref[...]
        wr = wrt_ref[...]
        br = brt_ref[...]
        for f in range(N_REAL):
            outk_ref[0, f] = (wr[:, f:f + 1] * kr[0, f][None, :]
                              + br[:, f:f + 1])
        ob = obs_ref[...]
        wo = wot_ref[...]
        bo = bot_ref[...]
        for f in range(N_OBS):
            outo_ref[0, f] = (wo[:, f:f + 1] * ob[f][None, :]
                              + bo[:, f:f + 1])

    out_k, out_o = pl.pallas_call(
        body,
        grid=grid,
        in_specs=[
            pl.BlockSpec(memory_space=pl.ANY),
            pl.BlockSpec((1, N_REAL, B), lambda t: (t, 0, 0)),
            pl.BlockSpec((N_OBS, B), lambda t: (0, t)),
            pl.BlockSpec((H, N_REAL), lambda t: (0, 0)),
            pl.BlockSpec((H, N_REAL), lambda t: (0, 0)),
            pl.BlockSpec((H, N_OBS), lambda t: (0, 0)),
            pl.BlockSpec((H, N_OBS), lambda t: (0, 0)),
        ],
        out_specs=[
            pl.BlockSpec((1, N_REAL, H, B), lambda t: (t, 0, 0, 0)),
            pl.BlockSpec((1, N_OBS, H, B), lambda t: (t, 0, 0, 0)),
        ],
        out_shape=[
            jax.ShapeDtypeStruct((T, N_KNOWN, H, B), jnp.float32),
            jax.ShapeDtypeStruct((T, N_OBS, H, B), jnp.float32),
        ],
        input_output_aliases={0: 0},
    )(kp_t, kr_t, obs_t, wrt, brt, wot, bot)
    return out_k, out_o


def kernel(static, known_real, known_categorical, observed,
           static_tables, known_cat_tables, W_real, b_real, W_obs, b_obs):
    kctbl_t = known_cat_tables.transpose(0, 2, 1)
    kc_t = known_categorical.transpose(1, 2, 0)
    kr_t = known_real.transpose(1, 2, 0)
    obs_t = observed.transpose(2, 1, 0).reshape(N_OBS, P)
    wrt = W_real.T
    brt = b_real.T
    wot = W_obs.T
    bot = b_obs.T

    st_tbl = static_tables.reshape(N_STATIC * STATIC_VOCAB, H)
    st_idx = (static + jnp.arange(N_STATIC, dtype=jnp.int32) * STATIC_VOCAB)
    st_idx = st_idx.reshape(S_ROWS // 128, 128)

    kp_t = _sc_cat(kctbl_t, kc_t)
    st_rows = _sc_static(st_tbl, st_idx)

    out_k, out_o = _tc_dense(kp_t, kr_t, obs_t, wrt, brt, wot, bot)

    static_emb = st_rows.reshape(B, N_STATIC, H)
    known_emb = out_k.transpose(3, 0, 1, 2)
    observed_emb = out_o.transpose(3, 0, 1, 2)
    return static_emb, known_emb, observed_emb


# speedup vs baseline: 3.1299x; 1.0070x over previous
import functools

import jax
import jax.numpy as jnp
from jax import lax
from jax.experimental import pallas as pl
from jax.experimental.pallas import tpu as pltpu
from jax.experimental.pallas import tpu_sc as plsc

B = 1024
T = 50
H = 64
N_STATIC = 8
STATIC_VOCAB = 100000
N_KNOWN_CAT = 4
KNOWN_CAT_VOCAB = 1000
N_REAL = 4
N_OBS = 6
N_KNOWN = N_REAL + N_KNOWN_CAT

P = B * T
S_ROWS = B * N_STATIC
NW = 32
SPW = S_ROWS // NW


def _sc_static(st_tbl2, stidx_hbm):
    mesh = plsc.VectorSubcoreMesh(core_axis_name="c", subcore_axis_name="s",
                                  num_cores=2, num_subcores=16)
    BQ = B // 4

    @functools.partial(
        pl.kernel,
        out_type=jax.ShapeDtypeStruct((N_STATIC, H, B), jnp.float32),
        mesh=mesh,
        compiler_params=pltpu.CompilerParams(use_tc_tiling_on_sc=True,
                                             needs_layout_passes=False),
        scratch_types=[
            pltpu.VMEM((2, 128), jnp.int32),
            pltpu.VMEM((2, 128), jnp.int32),
            pltpu.VMEM((BQ,), jnp.int32),
            pltpu.VMEM((BQ, 128), jnp.float32),
            pltpu.VMEM((H, BQ), jnp.float32),
            pltpu.SemaphoreType.DMA,
        ],
    )
    def k(tbl_ref, stidx_ref, out_ref, idx_v, ridx_v, par_v, rows_v,
          blk_v, sem):
        wid = lax.axis_index("s") * 2 + lax.axis_index("c")
        i = wid // 4
        q = wid % 4
        pltpu.sync_copy(stidx_ref.at[i, pl.ds(q * 2, 2)], idx_v)
        for g in range(2):
            for c in range(8):
                v = idx_v[g, pl.ds(c * 16, 16)]
                ridx_v[g, pl.ds(c * 16, 16)] = lax.shift_right_logical(v, 1)
                par_v[pl.ds(g * 128 + c * 16, 16)] = (v & 1) * H
        cps = [
            pltpu.async_copy(tbl_ref.at[ridx_v.at[g]],
                             rows_v.at[pl.ds(g * 128, 128)], sem)
            for g in range(2)
        ]
        for cp in cps:
            cp.wait()
        for g in range(BQ // 16):
            bv = jnp.arange(g * 16, g * 16 + 16, dtype=jnp.int32)
            p16 = par_v[pl.ds(g * 16, 16)]

            def h_body(h, carry):
                cv = p16 + h
                vals = plsc.load_gather(rows_v, [bv, cv])
                blk_v[h, pl.ds(g * 16, 16)] = vals
                return carry

            lax.fori_loop(0, H, h_body, 0, unroll=8)
        pltpu.sync_copy(blk_v, out_ref.at[i, :, pl.ds(q * BQ, BQ)])

    return k(st_tbl2, stidx_hbm)


def _sc_cat(kctbl_t, kc_t):
    mesh = plsc.VectorSubcoreMesh(core_axis_name="c", subcore_axis_name="s",
                                  num_cores=2, num_subcores=16)
    HS = 8

    VP = 1024

    @functools.partial(
        pl.kernel,
        out_type=jax.ShapeDtypeStruct((T, N_KNOWN, H, B), jnp.float32),
        mesh=mesh,
        compiler_params=pltpu.CompilerParams(use_tc_tiling_on_sc=True,
                                             needs_layout_passes=False),
        scratch_types=[
            pltpu.VMEM((HS, KNOWN_CAT_VOCAB), jnp.float32),
            pltpu.VMEM((HS * VP,), jnp.float32),
            pltpu.VMEM((2, B), jnp.int32),
            pltpu.VMEM((2, HS, B), jnp.float32),
            pltpu.SemaphoreType.DMA,
            pltpu.SemaphoreType.DMA,
        ],
    )
    def k(tbl_ref, idx_ref, out_ref, slab2_v, slab_v, idx_v, stage_v,
          sem_i, sem_w):
        wid = lax.axis_index("s") * 2 + lax.axis_index("c")
        fc = wid // 8
        h0 = (wid % 8) * HS
        pltpu.sync_copy(tbl_ref.at[fc, pl.ds(h0, HS)], slab2_v)
        for h in range(HS):
            def rearr(c, carry):
                off = c * 16
                slab_v[pl.ds(h * VP + off, 16)] = slab2_v[h, pl.ds(off, 16)]
                return carry
            lax.fori_loop(0, KNOWN_CAT_VOCAB // 16, rearr, 0, unroll=8)
            tail = KNOWN_CAT_VOCAB - 16
            slab_v[pl.ds(h * VP + tail, 16)] = slab2_v[h, pl.ds(tail, 16)]

        pltpu.make_async_copy(idx_ref.at[0, fc], idx_v.at[0], sem_i).start()

        def t_body(t, carry):
            buf = lax.rem(t, 2)
            pltpu.make_async_copy(idx_ref.at[t, fc], idx_v.at[buf],
                                  sem_i).wait()
            @pl.when(t + 1 < T)
            def _():
                pltpu.make_async_copy(idx_ref.at[t + 1, fc],
                                      idx_v.at[1 - buf], sem_i).start()

            for g in range(B // 16):
                iv = idx_v[buf, pl.ds(g * 16, 16)]
                for h in range(HS):
                    av = iv + (h * VP) if h else iv
                    vals = plsc.load_gather(slab_v, [av])
                    stage_v[buf, h, pl.ds(g * 16, 16)] = vals

            @pl.when(t >= 2)
            def _():
                pltpu.make_async_copy(
                    stage_v.at[buf],
                    out_ref.at[t - 2, N_REAL + fc, pl.ds(h0, HS)],
                    sem_w).wait()

            pltpu.make_async_copy(
                stage_v.at[buf],
                out_ref.at[t, N_REAL + fc, pl.ds(h0, HS)],
                sem_w).start()
            return carry

        lax.fori_loop(0, T, t_body, 0)
        for t in (T - 2, T - 1):
            pltpu.make_async_copy(
                stage_v.at[t % 2],
                out_ref.at[t, N_REAL + fc, pl.ds(h0, HS)],
                sem_w).wait()

    return k(kctbl_t, kc_t)


def _tc_dense(kp_t, kr_t, obs_t, wrt, brt, wot, bot):
    grid = (T,)

    def body(kp_ref, kr_ref, obs_ref, wrt_ref, brt_ref, wot_ref, bot_ref,
             outk_ref, outo_ref):
        del kp_ref
        kr = kr_ref[...]
        wr = wrt_ref[...]
        br = brt_ref[...]
        for f in range(N_REAL):
            outk_ref[0, f] = (wr[:, f:f + 1] * kr[0, f][None, :]
                              + br[:, f:f + 1])
        ob = obs_ref[...]
        wo = wot_ref[...]
        bo = bot_ref[...]
        for f in range(N_OBS):
            outo_ref[0, f] = (wo[:, f:f + 1] * ob[f][None, :]
                              + bo[:, f:f + 1])

    out_k, out_o = pl.pallas_call(
        body,
        grid=grid,
        in_specs=[
            pl.BlockSpec(memory_space=pl.ANY),
            pl.BlockSpec((1, N_REAL, B), lambda t: (t, 0, 0)),
            pl.BlockSpec((N_OBS, B), lambda t: (0, t)),
            pl.BlockSpec((H, N_REAL), lambda t: (0, 0)),
            pl.BlockSpec((H, N_REAL), lambda t: (0, 0)),
            pl.BlockSpec((H, N_OBS), lambda t: (0, 0)),
            pl.BlockSpec((H, N_OBS), lambda t: (0, 0)),
        ],
        out_specs=[
            pl.BlockSpec((1, N_REAL, H, B), lambda t: (t, 0, 0, 0)),
            pl.BlockSpec((1, N_OBS, H, B), lambda t: (t, 0, 0, 0)),
        ],
        out_shape=[
            jax.ShapeDtypeStruct((T, N_KNOWN, H, B), jnp.float32),
            jax.ShapeDtypeStruct((T, N_OBS, H, B), jnp.float32),
        ],
        input_output_aliases={0: 0},
    )(kp_t, kr_t, obs_t, wrt, brt, wot, bot)
    return out_k, out_o


def kernel(static, known_real, known_categorical, observed,
           static_tables, known_cat_tables, W_real, b_real, W_obs, b_obs):
    kctbl_t = known_cat_tables.transpose(0, 2, 1)
    kc_t = known_categorical.transpose(1, 2, 0)
    kr_t = known_real.transpose(1, 2, 0)
    obs_t = observed.transpose(2, 1, 0).reshape(N_OBS, P)
    wrt = W_real.T
    brt = b_real.T
    wot = W_obs.T
    bot = b_obs.T

    st_tbl2 = static_tables.reshape(N_STATIC * STATIC_VOCAB // 2, 2 * H)
    st_idx = (static + jnp.arange(N_STATIC, dtype=jnp.int32) * STATIC_VOCAB)
    st_idx = st_idx.T.reshape(N_STATIC, B // 128, 128)

    kp_t = _sc_cat(kctbl_t, kc_t)
    st_emb_t = _sc_static(st_tbl2, st_idx)

    out_k, out_o = _tc_dense(kp_t, kr_t, obs_t, wrt, brt, wot, bot)

    static_emb = st_emb_t.transpose(2, 0, 1)
    known_emb = out_k.transpose(3, 0, 1, 2)
    observed_emb = out_o.transpose(3, 0, 1, 2)
    return static_emb, known_emb, observed_emb


# speedup vs baseline: 3.4416x; 1.0996x over previous
import functools

import jax
import jax.numpy as jnp
from jax import lax
from jax.experimental import pallas as pl
from jax.experimental.pallas import tpu as pltpu
from jax.experimental.pallas import tpu_sc as plsc

B = 1024
T = 50
H = 64
N_STATIC = 8
STATIC_VOCAB = 100000
N_KNOWN_CAT = 4
KNOWN_CAT_VOCAB = 1000
N_REAL = 4
N_OBS = 6
N_KNOWN = N_REAL + N_KNOWN_CAT

P = B * T
S_ROWS = B * N_STATIC
NW = 32
SPW = S_ROWS // NW


def _sc_static(st_tbl2, stidx_hbm):
    mesh = plsc.VectorSubcoreMesh(core_axis_name="c", subcore_axis_name="s",
                                  num_cores=2, num_subcores=16)
    BQ = B // 4

    @functools.partial(
        pl.kernel,
        out_type=jax.ShapeDtypeStruct((N_STATIC, H, B), jnp.float32),
        mesh=mesh,
        compiler_params=pltpu.CompilerParams(use_tc_tiling_on_sc=True,
                                             needs_layout_passes=False),
        scratch_types=[
            pltpu.VMEM((2, 128), jnp.int32),
            pltpu.VMEM((BQ, 128), jnp.float32),
            pltpu.VMEM((H, BQ), jnp.float32),
            pltpu.SemaphoreType.DMA,
        ],
    )
    def k(tbl_ref, stidx_ref, out_ref, idx_v, rows_v, blk_v, sem):
        wid = lax.axis_index("s") * 2 + lax.axis_index("c")
        i = wid // 4
        q = wid % 4
        pltpu.sync_copy(stidx_ref.at[i, pl.ds(q * 2, 2)], idx_v)
        cps = [
            pltpu.async_copy(tbl_ref.at[idx_v.at[g]],
                             rows_v.at[pl.ds(g * 128, 128)], sem)
            for g in range(2)
        ]
        for cp in cps:
            cp.wait()
        for g in range(BQ // 16):
            bv = jnp.arange(g * 16, g * 16 + 16, dtype=jnp.int32)

            def h_body(h, carry):
                cv = jnp.full((16,), 0, jnp.int32) + h
                vals = plsc.load_gather(rows_v, [bv, cv])
                blk_v[h, pl.ds(g * 16, 16)] = vals
                return carry

            lax.fori_loop(0, H, h_body, 0, unroll=8)
        pltpu.sync_copy(blk_v, out_ref.at[i, :, pl.ds(q * BQ, BQ)])

    return k(st_tbl2, stidx_hbm)


def _sc_cat(kctbl_t, kc_t):
    mesh = plsc.VectorSubcoreMesh(core_axis_name="c", subcore_axis_name="s",
                                  num_cores=2, num_subcores=16)
    HS = 8

    VP = 1024

    @functools.partial(
        pl.kernel,
        out_type=jax.ShapeDtypeStruct((T, N_KNOWN, H, B), jnp.float32),
        mesh=mesh,
        compiler_params=pltpu.CompilerParams(use_tc_tiling_on_sc=True,
                                             needs_layout_passes=False),
        scratch_types=[
            pltpu.VMEM((HS, KNOWN_CAT_VOCAB), jnp.float32),
            pltpu.VMEM((HS * VP,), jnp.float32),
            pltpu.VMEM((2, B), jnp.int32),
            pltpu.VMEM((2, HS, B), jnp.float32),
            pltpu.SemaphoreType.DMA,
            pltpu.SemaphoreType.DMA,
        ],
    )
    def k(tbl_ref, idx_ref, out_ref, slab2_v, slab_v, idx_v, stage_v,
          sem_i, sem_w):
        wid = lax.axis_index("s") * 2 + lax.axis_index("c")
        fc = wid // 8
        h0 = (wid % 8) * HS
        pltpu.sync_copy(tbl_ref.at[fc, pl.ds(h0, HS)], slab2_v)
        for h in range(HS):
            def rearr(c, carry):
                off = c * 16
                slab_v[pl.ds(h * VP + off, 16)] = slab2_v[h, pl.ds(off, 16)]
                return carry
            lax.fori_loop(0, KNOWN_CAT_VOCAB // 16, rearr, 0, unroll=8)
            tail = KNOWN_CAT_VOCAB - 16
            slab_v[pl.ds(h * VP + tail, 16)] = slab2_v[h, pl.ds(tail, 16)]

        pltpu.make_async_copy(idx_ref.at[0, fc], idx_v.at[0], sem_i).start()

        def t_body(t, carry):
            buf = lax.rem(t, 2)
            pltpu.make_async_copy(idx_ref.at[t, fc], idx_v.at[buf],
                                  sem_i).wait()
            @pl.when(t + 1 < T)
            def _():
                pltpu.make_async_copy(idx_ref.at[t + 1, fc],
                                      idx_v.at[1 - buf], sem_i).start()

            for g in range(B // 16):
                iv = idx_v[buf, pl.ds(g * 16, 16)]
                for h in range(HS):
                    av = iv + (h * VP) if h else iv
                    vals = plsc.load_gather(slab_v, [av])
                    stage_v[buf, h, pl.ds(g * 16, 16)] = vals

            @pl.when(t >= 2)
            def _():
                pltpu.make_async_copy(
                    stage_v.at[buf],
                    out_ref.at[t - 2, N_REAL + fc, pl.ds(h0, HS)],
                    sem_w).wait()

            pltpu.make_async_copy(
                stage_v.at[buf],
                out_ref.at[t, N_REAL + fc, pl.ds(h0, HS)],
                sem_w).start()
            return carry

        lax.fori_loop(0, T, t_body, 0)
        for t in (T - 2, T - 1):
            pltpu.make_async_copy(
                stage_v.at[t % 2],
                out_ref.at[t, N_REAL + fc, pl.ds(h0, HS)],
                sem_w).wait()

    return k(kctbl_t, kc_t)


def _tc_dense(kp_t, kr_t, obs_t, wrt, brt, wot, bot):
    grid = (T,)

    def body(kp_ref, kr_ref, obs_ref, wrt_ref, brt_ref, wot_ref, bot_ref,
             outk_ref, outo_ref):
        del kp_ref
        kr = kr_ref[...]
        wr = wrt_ref[...]
        br = brt_ref[...]
        for f in range(N_REAL):
            outk_ref[0, f] = (wr[:, f:f + 1] * kr[0, f][None, :]
                              + br[:, f:f + 1])
        ob = obs_ref[...]
        wo = wot_ref[...]
        bo = bot_ref[...]
        for f in range(N_OBS):
            outo_ref[0, f] = (wo[:, f:f + 1] * ob[f][None, :]
                              + bo[:, f:f + 1])

    out_k, out_o = pl.pallas_call(
        body,
        grid=grid,
        in_specs=[
            pl.BlockSpec(memory_space=pl.ANY),
            pl.BlockSpec((1, N_REAL, B), lambda t: (t, 0, 0)),
            pl.BlockSpec((N_OBS, B), lambda t: (0, t)),
            pl.BlockSpec((H, N_REAL), lambda t: (0, 0)),
            pl.BlockSpec((H, N_REAL), lambda t: (0, 0)),
            pl.BlockSpec((H, N_OBS), lambda t: (0, 0)),
            pl.BlockSpec((H, N_OBS), lambda t: (0, 0)),
        ],
        out_specs=[
            pl.BlockSpec((1, N_REAL, H, B), lambda t: (t, 0, 0, 0)),
            pl.BlockSpec((1, N_OBS, H, B), lambda t: (t, 0, 0, 0)),
        ],
        out_shape=[
            jax.ShapeDtypeStruct((T, N_KNOWN, H, B), jnp.float32),
            jax.ShapeDtypeStruct((T, N_OBS, H, B), jnp.float32),
        ],
        input_output_aliases={0: 0},
    )(kp_t, kr_t, obs_t, wrt, brt, wot, bot)
    return out_k, out_o


def kernel(static, known_real, known_categorical, observed,
           static_tables, known_cat_tables, W_real, b_real, W_obs, b_obs):
    kctbl_t = known_cat_tables.transpose(0, 2, 1)
    kc_t = known_categorical.transpose(1, 2, 0)
    kr_t = known_real.transpose(1, 2, 0)
    obs_t = observed.transpose(2, 1, 0).reshape(N_OBS, P)
    wrt = W_real.T
    brt = b_real.T
    wot = W_obs.T
    bot = b_obs.T

    st_tbl2 = jnp.pad(static_tables, ((0, 0), (0, 0), (0, H)))
    st_tbl2 = st_tbl2.reshape(N_STATIC * STATIC_VOCAB, 2 * H)
    st_idx = (static + jnp.arange(N_STATIC, dtype=jnp.int32) * STATIC_VOCAB)
    st_idx = st_idx.T.reshape(N_STATIC, B // 128, 128)

    kp_t = _sc_cat(kctbl_t, kc_t)
    st_emb_t = _sc_static(st_tbl2, st_idx)

    out_k, out_o = _tc_dense(kp_t, kr_t, obs_t, wrt, brt, wot, bot)

    static_emb = st_emb_t.transpose(2, 0, 1)
    known_emb = out_k.transpose(3, 0, 1, 2)
    observed_emb = out_o.transpose(3, 0, 1, 2)
    return static_emb, known_emb, observed_emb


# speedup vs baseline: 5.2112x; 1.5142x over previous
import functools

import jax
import jax.numpy as jnp
from jax import lax
from jax.experimental import pallas as pl
from jax.experimental.pallas import tpu as pltpu
from jax.experimental.pallas import tpu_sc as plsc

B = 1024
T = 50
H = 64
N_STATIC = 8
STATIC_VOCAB = 100000
N_KNOWN_CAT = 4
KNOWN_CAT_VOCAB = 1000
N_REAL = 4
N_OBS = 6
N_KNOWN = N_REAL + N_KNOWN_CAT

P = B * T
S_ROWS = B * N_STATIC
NW = 32
SPW = S_ROWS // NW


def _sc_static(st_tbl2, stidx_hbm):
    mesh = plsc.VectorSubcoreMesh(core_axis_name="c", subcore_axis_name="s",
                                  num_cores=2, num_subcores=16)
    BQ = B // 4
    RING = 8

    @functools.partial(
        pl.kernel,
        out_type=jax.ShapeDtypeStruct((N_STATIC, H, B), jnp.float32),
        mesh=mesh,
        compiler_params=pltpu.CompilerParams(use_tc_tiling_on_sc=True,
                                             needs_layout_passes=False),
        scratch_types=[
            pltpu.VMEM((2, 128), jnp.int32),
            pltpu.VMEM((RING, H, 128), jnp.float32),
            pltpu.VMEM((BQ, H), jnp.float32),
            pltpu.VMEM((H, BQ), jnp.float32),
            pltpu.SemaphoreType.DMA,
        ],
    )
    def k(tbl_ref, stidx_ref, out_ref, idx_v, ring_v, rows_v, blk_v, sem):
        wid = lax.axis_index("s") * 2 + lax.axis_index("c")
        i = wid // 4
        q = wid % 4
        pltpu.sync_copy(stidx_ref.at[i, pl.ds(q * 2, 2)], idx_v)
        lanes16 = lax.iota(jnp.int32, 16)

        def scalar_idx(j):
            row = lax.shift_right_logical(j, 7)
            base = (lax.rem(j, 128) >> 4) << 4
            v16 = idx_v[row, pl.ds(base, 16)]
            return jnp.sum(jnp.where(lanes16 == (j & 15), v16, 0))

        def issue(j):
            v = scalar_idx(j)
            c = pl.multiple_of(
                lax.shift_right_logical(v, 7) * 128, 128)
            pltpu.make_async_copy(
                tbl_ref.at[i, :, pl.ds(c, 128)],
                ring_v.at[lax.rem(j, RING)], sem).start()

        def extract(jj):
            v = scalar_idx(jj)
            lane = v & 127
            slot = ring_v.at[lax.rem(jj, RING)]
            for hg in range(H // 16):
                hv = jnp.arange(hg * 16, hg * 16 + 16, dtype=jnp.int32)
                lv = jnp.full((16,), lane, jnp.int32)
                rows_v[jj, pl.ds(hg * 16, 16)] = plsc.load_gather(
                    slot, [hv, lv])

        def loop_body(j, carry):
            @pl.when(j < BQ)
            def _():
                issue(j)

            @pl.when(j >= RING)
            def _():
                pltpu.make_async_copy(
                    tbl_ref.at[i, :, pl.ds(0, 128)], ring_v.at[0],
                    sem).wait()
                extract(j - RING)
            return carry

        lax.fori_loop(0, BQ + RING, loop_body, 0)

        for g in range(BQ // 16):
            bv = jnp.arange(g * 16, g * 16 + 16, dtype=jnp.int32)

            def h_body(h, carry):
                cv = jnp.full((16,), 0, jnp.int32) + h
                blk_v[h, pl.ds(g * 16, 16)] = plsc.load_gather(
                    rows_v, [bv, cv])
                return carry

            lax.fori_loop(0, H, h_body, 0, unroll=8)
        pltpu.sync_copy(blk_v, out_ref.at[i, :, pl.ds(q * BQ, BQ)])

    return k(st_tbl2, stidx_hbm)


def _sc_cat(kctbl_t, kc_t):
    mesh = plsc.VectorSubcoreMesh(core_axis_name="c", subcore_axis_name="s",
                                  num_cores=2, num_subcores=16)
    HS = 8

    VP = 1024

    @functools.partial(
        pl.kernel,
        out_type=jax.ShapeDtypeStruct((T, N_KNOWN, H, B), jnp.float32),
        mesh=mesh,
        compiler_params=pltpu.CompilerParams(use_tc_tiling_on_sc=True,
                                             needs_layout_passes=False),
        scratch_types=[
            pltpu.VMEM((HS, KNOWN_CAT_VOCAB), jnp.float32),
            pltpu.VMEM((HS * VP,), jnp.float32),
            pltpu.VMEM((2, B), jnp.int32),
            pltpu.VMEM((2, HS, B), jnp.float32),
            pltpu.SemaphoreType.DMA,
            pltpu.SemaphoreType.DMA,
        ],
    )
    def k(tbl_ref, idx_ref, out_ref, slab2_v, slab_v, idx_v, stage_v,
          sem_i, sem_w):
        wid = lax.axis_index("s") * 2 + lax.axis_index("c")
        fc = wid // 8
        h0 = (wid % 8) * HS
        pltpu.sync_copy(tbl_ref.at[fc, pl.ds(h0, HS)], slab2_v)
        for h in range(HS):
            def rearr(c, carry):
                off = c * 16
                slab_v[pl.ds(h * VP + off, 16)] = slab2_v[h, pl.ds(off, 16)]
                return carry
            lax.fori_loop(0, KNOWN_CAT_VOCAB // 16, rearr, 0, unroll=8)
            tail = KNOWN_CAT_VOCAB - 16
            slab_v[pl.ds(h * VP + tail, 16)] = slab2_v[h, pl.ds(tail, 16)]

        pltpu.make_async_copy(idx_ref.at[0, fc], idx_v.at[0], sem_i).start()

        def t_body(t, carry):
            buf = lax.rem(t, 2)
            pltpu.make_async_copy(idx_ref.at[t, fc], idx_v.at[buf],
                                  sem_i).wait()
            @pl.when(t + 1 < T)
            def _():
                pltpu.make_async_copy(idx_ref.at[t + 1, fc],
                                      idx_v.at[1 - buf], sem_i).start()

            for g in range(B // 16):
                iv = idx_v[buf, pl.ds(g * 16, 16)]
                for h in range(HS):
                    av = iv + (h * VP) if h else iv
                    vals = plsc.load_gather(slab_v, [av])
                    stage_v[buf, h, pl.ds(g * 16, 16)] = vals

            @pl.when(t >= 2)
            def _():
                pltpu.make_async_copy(
                    stage_v.at[buf],
                    out_ref.at[t - 2, N_REAL + fc, pl.ds(h0, HS)],
                    sem_w).wait()

            pltpu.make_async_copy(
                stage_v.at[buf],
                out_ref.at[t, N_REAL + fc, pl.ds(h0, HS)],
                sem_w).start()
            return carry

        lax.fori_loop(0, T, t_body, 0)
        for t in (T - 2, T - 1):
            pltpu.make_async_copy(
                stage_v.at[t % 2],
                out_ref.at[t, N_REAL + fc, pl.ds(h0, HS)],
                sem_w).wait()

    return k(kctbl_t, kc_t)


def _tc_dense(kp_t, kr_t, obs_t, wrt, brt, wot, bot):
    grid = (T,)

    def body(kp_ref, kr_ref, obs_ref, wrt_ref, brt_ref, wot_ref, bot_ref,
             outk_ref, outo_ref):
        del kp_ref
        kr = kr_ref[...]
        wr = wrt_ref[...]
        br = brt_ref[...]
        for f in range(N_REAL):
            outk_ref[0, f] = (wr[:, f:f + 1] * kr[0, f][None, :]
                              + br[:, f:f + 1])
        ob = obs_ref[...]
        wo = wot_ref[...]
        bo = bot_ref[...]
        for f in range(N_OBS):
            outo_ref[0, f] = (wo[:, f:f + 1] * ob[f][None, :]
                              + bo[:, f:f + 1])

    out_k, out_o = pl.pallas_call(
        body,
        grid=grid,
        in_specs=[
            pl.BlockSpec(memory_space=pl.ANY),
            pl.BlockSpec((1, N_REAL, B), lambda t: (t, 0, 0)),
            pl.BlockSpec((N_OBS, B), lambda t: (0, t)),
            pl.BlockSpec((H, N_REAL), lambda t: (0, 0)),
            pl.BlockSpec((H, N_REAL), lambda t: (0, 0)),
            pl.BlockSpec((H, N_OBS), lambda t: (0, 0)),
            pl.BlockSpec((H, N_OBS), lambda t: (0, 0)),
        ],
        out_specs=[
            pl.BlockSpec((1, N_REAL, H, B), lambda t: (t, 0, 0, 0)),
            pl.BlockSpec((1, N_OBS, H, B), lambda t: (t, 0, 0, 0)),
        ],
        out_shape=[
            jax.ShapeDtypeStruct((T, N_KNOWN, H, B), jnp.float32),
            jax.ShapeDtypeStruct((T, N_OBS, H, B), jnp.float32),
        ],
        input_output_aliases={0: 0},
    )(kp_t, kr_t, obs_t, wrt, brt, wot, bot)
    return out_k, out_o


def kernel(static, known_real, known_categorical, observed,
           static_tables, known_cat_tables, W_real, b_real, W_obs, b_obs):
    kctbl_t = known_cat_tables.transpose(0, 2, 1)
    kc_t = known_categorical.transpose(1, 2, 0)
    kr_t = known_real.transpose(1, 2, 0)
    obs_t = observed.transpose(2, 1, 0).reshape(N_OBS, P)
    wrt = W_real.T
    brt = b_real.T
    wot = W_obs.T
    bot = b_obs.T

    st_tbl2 = static_tables.transpose(0, 2, 1)
    st_idx = static.T.reshape(N_STATIC, B // 128, 128)

    kp_t = _sc_cat(kctbl_t, kc_t)
    st_emb_t = _sc_static(st_tbl2, st_idx)

    out_k, out_o = _tc_dense(kp_t, kr_t, obs_t, wrt, brt, wot, bot)

    static_emb = st_emb_t.transpose(2, 0, 1)
    known_emb = out_k.transpose(3, 0, 1, 2)
    observed_emb = out_o.transpose(3, 0, 1, 2)
    return static_emb, known_emb, observed_emb


# speedup vs baseline: 5.2355x; 1.0047x over previous
import functools

import jax
import jax.numpy as jnp
from jax import lax
from jax.experimental import pallas as pl
from jax.experimental.pallas import tpu as pltpu
from jax.experimental.pallas import tpu_sc as plsc

B = 1024
T = 50
H = 64
N_STATIC = 8
STATIC_VOCAB = 100000
N_KNOWN_CAT = 4
KNOWN_CAT_VOCAB = 1000
N_REAL = 4
N_OBS = 6
N_KNOWN = N_REAL + N_KNOWN_CAT

P = B * T
S_ROWS = B * N_STATIC
NW = 32
SPW = S_ROWS // NW


def _sc_static(st_tbl2, stidx_hbm):
    mesh = plsc.VectorSubcoreMesh(core_axis_name="c", subcore_axis_name="s",
                                  num_cores=2, num_subcores=16)
    BQ = B // 4
    RING = 8

    @functools.partial(
        pl.kernel,
        out_type=jax.ShapeDtypeStruct((N_STATIC, H, B), jnp.float32),
        mesh=mesh,
        compiler_params=pltpu.CompilerParams(use_tc_tiling_on_sc=True,
                                             needs_layout_passes=False),
        scratch_types=[
            pltpu.VMEM((2, 128), jnp.int32),
            pltpu.VMEM((RING, H, 128), jnp.float32),
            pltpu.VMEM((BQ, H), jnp.float32),
            pltpu.VMEM((H, BQ), jnp.float32),
            pltpu.SemaphoreType.DMA,
        ],
    )
    def k(tbl_ref, stidx_ref, out_ref, idx_v, ring_v, rows_v, blk_v, sem):
        wid = lax.axis_index("s") * 2 + lax.axis_index("c")
        i = wid // 4
        q = wid % 4
        pltpu.sync_copy(stidx_ref.at[i, pl.ds(q * 2, 2)], idx_v)
        lanes16 = lax.iota(jnp.int32, 16)

        def scalar_idx(j):
            row = lax.shift_right_logical(j, 7)
            base = (lax.rem(j, 128) >> 4) << 4
            v16 = idx_v[row, pl.ds(base, 16)]
            return jnp.sum(jnp.where(lanes16 == (j & 15), v16, 0))

        def issue(j):
            v = scalar_idx(j)
            c = pl.multiple_of(
                lax.shift_right_logical(v, 7) * 128, 128)
            pltpu.make_async_copy(
                tbl_ref.at[i, :, pl.ds(c, 128)],
                ring_v.at[lax.rem(j, RING)], sem).start()

        def extract(jj):
            v = scalar_idx(jj)
            lane = v & 127
            slot = ring_v.at[lax.rem(jj, RING)]
            for hg in range(H // 16):
                hv = jnp.arange(hg * 16, hg * 16 + 16, dtype=jnp.int32)
                lv = jnp.full((16,), lane, jnp.int32)
                rows_v[jj, pl.ds(hg * 16, 16)] = plsc.load_gather(
                    slot, [hv, lv])

        def loop_body(j, carry):
            @pl.when(j >= RING)
            def _():
                pltpu.make_async_copy(
                    tbl_ref.at[i, :, pl.ds(0, 128)], ring_v.at[0],
                    sem).wait()
                extract(j - RING)

            @pl.when(j < BQ)
            def _():
                issue(j)
            return carry

        lax.fori_loop(0, BQ + RING, loop_body, 0)

        for g in range(BQ // 16):
            bv = jnp.arange(g * 16, g * 16 + 16, dtype=jnp.int32)

            def h_body(h, carry):
                cv = jnp.full((16,), 0, jnp.int32) + h
                blk_v[h, pl.ds(g * 16, 16)] = plsc.load_gather(
                    rows_v, [bv, cv])
                return carry

            lax.fori_loop(0, H, h_body, 0, unroll=8)
        pltpu.sync_copy(blk_v, out_ref.at[i, :, pl.ds(q * BQ, BQ)])

    return k(st_tbl2, stidx_hbm)


def _sc_cat(kctbl_t, kc_t):
    mesh = plsc.VectorSubcoreMesh(core_axis_name="c", subcore_axis_name="s",
                                  num_cores=2, num_subcores=16)
    HS = 8

    VP = 1024

    @functools.partial(
        pl.kernel,
        out_type=jax.ShapeDtypeStruct((T, N_KNOWN, H, B), jnp.float32),
        mesh=mesh,
        compiler_params=pltpu.CompilerParams(use_tc_tiling_on_sc=True,
                                             needs_layout_passes=False),
        scratch_types=[
            pltpu.VMEM((HS, KNOWN_CAT_VOCAB), jnp.float32),
            pltpu.VMEM((HS * VP,), jnp.float32),
            pltpu.VMEM((2, B), jnp.int32),
            pltpu.VMEM((2, HS, B), jnp.float32),
            pltpu.SemaphoreType.DMA,
            pltpu.SemaphoreType.DMA,
        ],
    )
    def k(tbl_ref, idx_ref, out_ref, slab2_v, slab_v, idx_v, stage_v,
          sem_i, sem_w):
        wid = lax.axis_index("s") * 2 + lax.axis_index("c")
        fc = wid // 8
        h0 = (wid % 8) * HS
        pltpu.sync_copy(tbl_ref.at[fc, pl.ds(h0, HS)], slab2_v)
        for h in range(HS):
            def rearr(c, carry):
                off = c * 16
                slab_v[pl.ds(h * VP + off, 16)] = slab2_v[h, pl.ds(off, 16)]
                return carry
            lax.fori_loop(0, KNOWN_CAT_VOCAB // 16, rearr, 0, unroll=8)
            tail = KNOWN_CAT_VOCAB - 16
            slab_v[pl.ds(h * VP + tail, 16)] = slab2_v[h, pl.ds(tail, 16)]

        pltpu.make_async_copy(idx_ref.at[0, fc], idx_v.at[0], sem_i).start()

        def t_body(t, carry):
            buf = lax.rem(t, 2)
            pltpu.make_async_copy(idx_ref.at[t, fc], idx_v.at[buf],
                                  sem_i).wait()
            @pl.when(t + 1 < T)
            def _():
                pltpu.make_async_copy(idx_ref.at[t + 1, fc],
                                      idx_v.at[1 - buf], sem_i).start()

            for g in range(B // 16):
                iv = idx_v[buf, pl.ds(g * 16, 16)]
                for h in range(HS):
                    av = iv + (h * VP) if h else iv
                    vals = plsc.load_gather(slab_v, [av])
                    stage_v[buf, h, pl.ds(g * 16, 16)] = vals

            @pl.when(t >= 2)
            def _():
                pltpu.make_async_copy(
                    stage_v.at[buf],
                    out_ref.at[t - 2, N_REAL + fc, pl.ds(h0, HS)],
                    sem_w).wait()

            pltpu.make_async_copy(
                stage_v.at[buf],
                out_ref.at[t, N_REAL + fc, pl.ds(h0, HS)],
                sem_w).start()
            return carry

        lax.fori_loop(0, T, t_body, 0)
        for t in (T - 2, T - 1):
            pltpu.make_async_copy(
                stage_v.at[t % 2],
                out_ref.at[t, N_REAL + fc, pl.ds(h0, HS)],
                sem_w).wait()

    return k(kctbl_t, kc_t)


def _tc_dense(kp_t, kr_t, obs_t, wrt, brt, wot, bot):
    grid = (T,)

    def body(kp_ref, kr_ref, obs_ref, wrt_ref, brt_ref, wot_ref, bot_ref,
             outk_ref, outo_ref):
        del kp_ref
        kr = kr_ref[...]
        wr = wrt_ref[...]
        br = brt_ref[...]
        for f in range(N_REAL):
            outk_ref[0, f] = (wr[:, f:f + 1] * kr[0, f][None, :]
                              + br[:, f:f + 1])
        ob = obs_ref[...]
        wo = wot_ref[...]
        bo = bot_ref[...]
        for f in range(N_OBS):
            outo_ref[0, f] = (wo[:, f:f + 1] * ob[f][None, :]
                              + bo[:, f:f + 1])

    out_k, out_o = pl.pallas_call(
        body,
        grid=grid,
        in_specs=[
            pl.BlockSpec(memory_space=pl.ANY),
            pl.BlockSpec((1, N_REAL, B), lambda t: (t, 0, 0)),
            pl.BlockSpec((N_OBS, B), lambda t: (0, t)),
            pl.BlockSpec((H, N_REAL), lambda t: (0, 0)),
            pl.BlockSpec((H, N_REAL), lambda t: (0, 0)),
            pl.BlockSpec((H, N_OBS), lambda t: (0, 0)),
            pl.BlockSpec((H, N_OBS), lambda t: (0, 0)),
        ],
        out_specs=[
            pl.BlockSpec((1, N_REAL, H, B), lambda t: (t, 0, 0, 0)),
            pl.BlockSpec((1, N_OBS, H, B), lambda t: (t, 0, 0, 0)),
        ],
        out_shape=[
            jax.ShapeDtypeStruct((T, N_KNOWN, H, B), jnp.float32),
            jax.ShapeDtypeStruct((T, N_OBS, H, B), jnp.float32),
        ],
        input_output_aliases={0: 0},
    )(kp_t, kr_t, obs_t, wrt, brt, wot, bot)
    return out_k, out_o


def kernel(static, known_real, known_categorical, observed,
           static_tables, known_cat_tables, W_real, b_real, W_obs, b_obs):
    kctbl_t = known_cat_tables.transpose(0, 2, 1)
    kc_t = known_categorical.transpose(1, 2, 0)
    kr_t = known_real.transpose(1, 2, 0)
    obs_t = observed.transpose(2, 1, 0).reshape(N_OBS, P)
    wrt = W_real.T
    brt = b_real.T
    wot = W_obs.T
    bot = b_obs.T

    st_tbl2 = static_tables.transpose(0, 2, 1)
    st_idx = static.T.reshape(N_STATIC, B // 128, 128)

    kp_t = _sc_cat(kctbl_t, kc_t)
    st_emb_t = _sc_static(st_tbl2, st_idx)

    out_k, out_o = _tc_dense(kp_t, kr_t, obs_t, wrt, brt, wot, bot)

    static_emb = st_emb_t.transpose(2, 0, 1)
    known_emb = out_k.transpose(3, 0, 1, 2)
    observed_emb = out_o.transpose(3, 0, 1, 2)
    return static_emb, known_emb, observed_emb


# speedup vs baseline: 5.3327x; 1.0186x over previous
import functools

import jax
import jax.numpy as jnp
from jax import lax
from jax.experimental import pallas as pl
from jax.experimental.pallas import tpu as pltpu
from jax.experimental.pallas import tpu_sc as plsc

B = 1024
T = 50
H = 64
N_STATIC = 8
STATIC_VOCAB = 100000
N_KNOWN_CAT = 4
KNOWN_CAT_VOCAB = 1000
N_REAL = 4
N_OBS = 6
N_KNOWN = N_REAL + N_KNOWN_CAT

P = B * T
NW = 32


def _sc_gathers(kctbl_t, kc_t, st_tbl_t, st_idx):
    mesh = plsc.VectorSubcoreMesh(core_axis_name="c", subcore_axis_name="s",
                                  num_cores=2, num_subcores=16)
    HS = 8
    VP = 1024
    BQ = B // 4
    RING = 5
    SPT = 6

    @functools.partial(
        pl.kernel,
        out_type=(
            jax.ShapeDtypeStruct((T, N_KNOWN, H, B), jnp.float32),
            jax.ShapeDtypeStruct((N_STATIC, H, B), jnp.float32),
        ),
        mesh=mesh,
        compiler_params=pltpu.CompilerParams(use_tc_tiling_on_sc=True,
                                             needs_layout_passes=False),
        scratch_types=[
            pltpu.VMEM((HS, KNOWN_CAT_VOCAB), jnp.float32),
            pltpu.VMEM((HS * VP,), jnp.float32),
            pltpu.VMEM((2, B), jnp.int32),
            pltpu.VMEM((2, HS, B), jnp.float32),
            pltpu.VMEM((2, 128), jnp.int32),
            pltpu.VMEM((RING, H, 128), jnp.float32),
            pltpu.VMEM((BQ, H), jnp.float32),
            pltpu.VMEM((H, BQ), jnp.float32),
            pltpu.SemaphoreType.DMA,
            pltpu.SemaphoreType.DMA,
            pltpu.SemaphoreType.DMA,
        ],
    )
    def k(tbl_ref, idx_ref, sttbl_ref, stidx_ref, out_ref, stout_ref,
          slab2_v, slab_v, cidx_v, cstage_v, sidx_v, ring_v, srows_v, sblk_v,
          sem_i, sem_w, sem_s):
        wid = lax.axis_index("s") * 2 + lax.axis_index("c")
        fc = wid // 8
        h0 = (wid % 8) * HS
        si = wid // 4
        sq = wid % 4

        pltpu.sync_copy(tbl_ref.at[fc, pl.ds(h0, HS)], slab2_v)
        for h in range(HS):
            def rearr(c, carry):
                off = c * 16
                slab_v[pl.ds(h * VP + off, 16)] = slab2_v[h, pl.ds(off, 16)]
                return carry
            lax.fori_loop(0, KNOWN_CAT_VOCAB // 16, rearr, 0, unroll=8)
            tail = KNOWN_CAT_VOCAB - 16
            slab_v[pl.ds(h * VP + tail, 16)] = slab2_v[h, pl.ds(tail, 16)]
        pltpu.sync_copy(stidx_ref.at[si, pl.ds(sq * 2, 2)], sidx_v)
        pltpu.make_async_copy(idx_ref.at[0, fc], cidx_v.at[0], sem_i).start()
        lanes16 = lax.iota(jnp.int32, 16)

        def scalar_idx(j):
            row = lax.shift_right_logical(j, 7)
            base = (lax.rem(j, 128) >> 4) << 4
            v16 = sidx_v[row, pl.ds(base, 16)]
            return jnp.sum(jnp.where(lanes16 == (j & 15), v16, 0))

        def s_issue(j):
            v = scalar_idx(j)
            c = pl.multiple_of(lax.shift_right_logical(v, 7) * 128, 128)
            pltpu.make_async_copy(
                sttbl_ref.at[si, :, pl.ds(c, 128)],
                ring_v.at[lax.rem(j, RING)], sem_s).start()

        def s_extract(jj):
            v = scalar_idx(jj)
            lane = v & 127
            slot = ring_v.at[lax.rem(jj, RING)]
            for hg in range(H // 16):
                hv = jnp.arange(hg * 16, hg * 16 + 16, dtype=jnp.int32)
                lv = jnp.full((16,), lane, jnp.int32)
                srows_v[jj, pl.ds(hg * 16, 16)] = plsc.load_gather(
                    slot, [hv, lv])

        def t_body(t, carry):
            buf = lax.rem(t, 2)
            pltpu.make_async_copy(idx_ref.at[t, fc], cidx_v.at[buf],
                                  sem_i).wait()
            @pl.when(t + 1 < T)
            def _():
                pltpu.make_async_copy(idx_ref.at[t + 1, fc],
                                      cidx_v.at[1 - buf], sem_i).start()

            for g in range(B // 16):
                iv = cidx_v[buf, pl.ds(g * 16, 16)]
                for h in range(HS):
                    av = iv + (h * VP) if h else iv
                    vals = plsc.load_gather(slab_v, [av])
                    cstage_v[buf, h, pl.ds(g * 16, 16)] = vals

            for k_ in range(SPT):
                j = t * SPT + k_

                @pl.when((j >= RING) & (j < BQ + RING))
                def _():
                    pltpu.make_async_copy(
                        sttbl_ref.at[si, :, pl.ds(0, 128)], ring_v.at[0],
                        sem_s).wait()
                    s_extract(j - RING)

                @pl.when(j < BQ)
                def _():
                    s_issue(j)

            @pl.when(t >= 2)
            def _():
                pltpu.make_async_copy(
                    cstage_v.at[buf],
                    out_ref.at[t - 2, N_REAL + fc, pl.ds(h0, HS)],
                    sem_w).wait()

            pltpu.make_async_copy(
                cstage_v.at[buf],
                out_ref.at[t, N_REAL + fc, pl.ds(h0, HS)],
                sem_w).start()
            return carry

        lax.fori_loop(0, T, t_body, 0)
        for t in (T - 2, T - 1):
            pltpu.make_async_copy(
                cstage_v.at[t % 2],
                out_ref.at[t, N_REAL + fc, pl.ds(h0, HS)],
                sem_w).wait()

        for g in range(BQ // 16):
            bv = jnp.arange(g * 16, g * 16 + 16, dtype=jnp.int32)

            def h_body(h, carry):
                cv = jnp.full((16,), 0, jnp.int32) + h
                sblk_v[h, pl.ds(g * 16, 16)] = plsc.load_gather(
                    srows_v, [bv, cv])
                return carry

            lax.fori_loop(0, H, h_body, 0, unroll=8)
        pltpu.sync_copy(sblk_v, stout_ref.at[si, :, pl.ds(sq * BQ, BQ)])

    return k(kctbl_t, kc_t, st_tbl_t, st_idx)


def _tc_dense(kp_t, kr_t, obs_t, wrt, brt, wot, bot):
    grid = (T,)

    def body(kp_ref, kr_ref, obs_ref, wrt_ref, brt_ref, wot_ref, bot_ref,
             outk_ref, outo_ref):
        del kp_ref
        kr = kr_ref[...]
        wr = wrt_ref[...]
        br = brt_ref[...]
        for f in range(N_REAL):
            outk_ref[0, f] = (wr[:, f:f + 1] * kr[0, f][None, :]
                              + br[:, f:f + 1])
        ob = obs_ref[...]
        wo = wot_ref[...]
        bo = bot_ref[...]
        for f in range(N_OBS):
            outo_ref[0, f] = (wo[:, f:f + 1] * ob[f][None, :]
                              + bo[:, f:f + 1])

    out_k, out_o = pl.pallas_call(
        body,
        grid=grid,
        in_specs=[
            pl.BlockSpec(memory_space=pl.ANY),
            pl.BlockSpec((1, N_REAL, B), lambda t: (t, 0, 0)),
            pl.BlockSpec((N_OBS, B), lambda t: (0, t)),
            pl.BlockSpec((H, N_REAL), lambda t: (0, 0)),
            pl.BlockSpec((H, N_REAL), lambda t: (0, 0)),
            pl.BlockSpec((H, N_OBS), lambda t: (0, 0)),
            pl.BlockSpec((H, N_OBS), lambda t: (0, 0)),
        ],
        out_specs=[
            pl.BlockSpec((1, N_REAL, H, B), lambda t: (t, 0, 0, 0)),
            pl.BlockSpec((1, N_OBS, H, B), lambda t: (t, 0, 0, 0)),
        ],
        out_shape=[
            jax.ShapeDtypeStruct((T, N_KNOWN, H, B), jnp.float32),
            jax.ShapeDtypeStruct((T, N_OBS, H, B), jnp.float32),
        ],
        input_output_aliases={0: 0},
    )(kp_t, kr_t, obs_t, wrt, brt, wot, bot)
    return out_k, out_o


def kernel(static, known_real, known_categorical, observed,
           static_tables, known_cat_tables, W_real, b_real, W_obs, b_obs):
    kctbl_t = known_cat_tables.transpose(0, 2, 1)
    kc_t = known_categorical.transpose(1, 2, 0)
    kr_t = known_real.transpose(1, 2, 0)
    obs_t = observed.transpose(2, 1, 0).reshape(N_OBS, P)
    wrt = W_real.T
    brt = b_real.T
    wot = W_obs.T
    bot = b_obs.T

    st_tbl_t = static_tables.transpose(0, 2, 1)
    st_idx = static.T.reshape(N_STATIC, B // 128, 128)

    kp_t, st_emb_t = _sc_gathers(kctbl_t, kc_t, st_tbl_t, st_idx)

    out_k, out_o = _tc_dense(kp_t, kr_t, obs_t, wrt, brt, wot, bot)

    static_emb = st_emb_t.transpose(2, 0, 1)
    known_emb = out_k.transpose(3, 0, 1, 2)
    observed_emb = out_o.transpose(3, 0, 1, 2)
    return static_emb, known_emb, observed_emb
